# baseline (speedup 1.0000x reference)
import functools

import jax
import jax.numpy as jnp
from jax.experimental import pallas as pl

_B = 1024
_MOL_N = 40
_RING_N = 12
_NF = 128
_EF = 4
_H = 128
_G = 128
_F = 16
_NCLQ = 64
_MLP_H = 512
_STEPS = 2

_BB = 32
_RING_PACK = 4

_SELU_S = 1.0507009873554805
_SELU_A = 1.6732632423543772

bf16 = jnp.bfloat16
f32 = jnp.float32


def _selu(x):
    return _SELU_S * jnp.where(x > 0, x, _SELU_A * (jnp.exp(x) - 1.0))


def _dot(x, w):
    return jax.lax.dot_general(x, w, (((1,), (0,)), ((), ())),
                               preferred_element_type=f32)


def _mmk(x, wstk):
    xb = x.astype(bf16)
    return _dot(jnp.concatenate([xb, xb], axis=1), wstk)


def _mm2(xb, wcat, dout):
    y = _dot(xb, wcat)
    return y[:, :dout] + y[:, dout:]


def _hilo_lanes(x):
    hi = x.astype(bf16)
    lo = (x - hi.astype(f32)).astype(bf16)
    return jnp.concatenate([hi, lo], axis=-1)


def _stack_hilo(w):
    hi = w.astype(bf16)
    lo = (w - hi.astype(f32)).astype(bf16)
    return jnp.concatenate([hi, lo], axis=0)


def _cat_hilo(w):
    hi = w.astype(bf16)
    lo = (w - hi.astype(f32)).astype(bf16)
    return jnp.concatenate([hi, lo], axis=1)


def _const_spec(arr):
    nd = arr.ndim
    return pl.BlockSpec(arr.shape, lambda i, _nd=nd: (0,) * _nd)


def _ggnn_compute(n, bb, pack, x, e3, wrefs):
    (wemb, bemb, wmsg, wzr, bzr, whc, bhc, gw, gb, ew, eb) = wrefs
    h = jnp.tanh(_mmk(x, wemb[...]) + bemb[...])
    ek = _EF * n
    if pack > 1:
        e4 = e3.reshape(bb // pack, pack, n, ek)
        bands = []
        for g in range(pack):
            pieces = []
            if g:
                pieces.append(jnp.zeros((bb // pack, n, g * ek), bf16))
            pieces.append(e4[:, g])
            if pack - 1 - g:
                pieces.append(
                    jnp.zeros((bb // pack, n, (pack - 1 - g) * ek), bf16))
            bands.append(jnp.concatenate(pieces, axis=2))
        eblk = jnp.concatenate(bands, axis=1)
    for _ in range(_STEPS):
        y = _mmk(h, wmsg[...])
        cats = [_hilo_lanes(y[:, _H * e: _H * (e + 1)]).reshape(bb, n, 2 * _H)
                for e in range(_EF)]
        whm3 = jnp.concatenate(cats, axis=1)
        ms = []
        if pack > 1:
            wblk = whm3.reshape(bb // pack, pack * ek, 2 * _H)
            for q in range(bb // pack):
                mq = _dot(eblk[q], wblk[q])
                ms.append(mq[:, :_H] + mq[:, _H:])
        else:
            for g in range(bb):
                mg_ = _dot(e3[g], whm3[g])
                ms.append(mg_[:, :_H] + mg_[:, _H:])
        m = jnp.concatenate(ms, axis=0)
        zr = jax.nn.sigmoid(
            _mmk(jnp.concatenate([m, h], axis=1), wzr[...]) + bzr[...])
        z, r = zr[:, :_H], zr[:, _H:]
        x2 = jnp.concatenate([m, r * h], axis=1).astype(bf16)
        hc = jnp.tanh(_mm2(x2, whc[...], _H) + bhc[...])
        h = (1.0 - z) * h + z * hc
    hb = h.astype(bf16)
    gate = jax.nn.sigmoid(_dot(jnp.concatenate([hb, x], axis=1), gw[...])
                          + gb[...])
    emb = _dot(hb, ew[...]) + eb[...]
    pooled = (gate * emb).reshape(bb, n, _G).sum(axis=1)
    return pooled, hb


def _mega_body(bb, nodesm_ref, edgesm_ref, nodesr_ref, edgesr_ref, ft_ref,
               *rest):
    wm = rest[0:11]
    wr = rest[11:22]
    (a0w, a0b, a1w, a1b, a2w, a2b,
     c0w, c0b, c1w, c1b, c2w, c2b) = rest[22:34]
    (awn, awr, awm, awf, ab0, aw1, ab1, aw2, ab2) = rest[34:43]
    (cwn, cwr, cwm, cwf, cb0, cw1, cb1, cw2, cb2) = rest[43:52]
    (twr, twf, tb0, tw1, tb1, tw2, tb2) = rest[52:59]
    add_ref, conn_ref, term_ref = rest[59:62]

    xm = nodesm_ref[...].astype(bf16)
    xr = nodesr_ref[...].astype(bf16)
    ftb = ft_ref[...].astype(bf16)

    mg, _ = _ggnn_compute(_MOL_N, bb, 1, xm, edgesm_ref[...], wm)
    rg, hr = _ggnn_compute(_RING_N, bb, _RING_PACK, xr, edgesr_ref[...], wr)
    mgb = mg.astype(bf16)
    rgb = rg.astype(bf16)

    def node_mlp(w0, b0, w1, b1, w2, b2):
        t = _selu(_dot(hr, w0[...]) + b0[...])
        t = _selu(_dot(t.astype(bf16), w1[...]) + b1[...])
        return (_dot(t.astype(bf16), w2[...]) + b2[...]) \
            .astype(bf16).reshape(bb, _RING_N, _F)

    a1 = node_mlp(a0w, a0b, a1w, a1b, a2w, a2b)
    c1 = node_mlp(c0w, c0b, c1w, c1b, c2w, c2b)

    def head(x1, wn, wrg, wmg, wf, b0, w1, b1, w2, b2):
        t = _dot(rgb, wrg[...]) + _dot(ftb, wf[...])
        if wmg is not None:
            t = t + _dot(mgb, wmg[...])
        if x1 is not None:
            for nn in range(_RING_N):
                t = t + _dot(x1[:, nn, :], wn[nn])
        t = _selu(t + b0[...])
        t = _selu(_dot(t.astype(bf16), w1[...]) + b1[...])
        return _dot(t.astype(bf16), w2[...]) + b2[...]

    add_ref[...] = head(a1, awn, awr, awm, awf, ab0, aw1, ab1, aw2, ab2)
    conn_ref[...] = head(c1, cwn, cwr, cwm, cwf, cb0, cw1, cb1, cw2, cb2)
    term_ref[...] = head(None, None, twr, None, twf, tb0, tw1, tb1, tw2, tb2)


def _prep_ggnn(p):
    wemb = _stack_hilo(p['embed'][0])
    bemb = p['embed'][1].reshape(1, -1)
    wmsg = jnp.concatenate([_stack_hilo(p['msg_W'][e]) for e in range(_EF)],
                           axis=1)
    wzr_f = jnp.concatenate(
        [jnp.concatenate([p['Wz'][0], p['Uz'][0]], axis=0),
         jnp.concatenate([p['Wr'][0], p['Ur'][0]], axis=0)], axis=1)
    wzr = _stack_hilo(wzr_f)
    bzr = jnp.concatenate([p['Wz'][1] + p['Uz'][1],
                           p['Wr'][1] + p['Ur'][1]]).reshape(1, -1)
    whc = _cat_hilo(jnp.concatenate([p['Wh'][0], p['Uh'][0]], axis=0))
    bhc = (p['Wh'][1] + p['Uh'][1]).reshape(1, -1)
    gw = p['gate'][0].astype(bf16)
    gb = p['gate'][1].reshape(1, -1)
    ew = p['emb'][0].astype(bf16)
    eb = p['emb'][1].reshape(1, -1)
    return [wemb, bemb, wmsg, wzr, bzr, whc, bhc, gw, gb, ew, eb]


def _prep_mlp(layers, pad_out=None):
    out = []
    for i, (w, b) in enumerate(layers):
        if pad_out is not None and i == len(layers) - 1:
            w = jnp.pad(w, ((0, 0), (0, pad_out - w.shape[1])))
            b = jnp.pad(b, ((0, pad_out - b.shape[0]),))
        out += [w.astype(bf16), b.reshape(1, -1)]
    return out


def _prep_head(layers, segs, pad_out=None):
    (w0, b0) = layers[0]
    out = []
    off = 0
    for name, sz in segs:
        piece = w0[off:off + sz].astype(bf16)
        if name == 'nodes':
            piece = piece.reshape(_RING_N, _F, _MLP_H)
        out.append(piece)
        off += sz
    out.append(b0.reshape(1, -1))
    out += _prep_mlp(layers[1:], pad_out=pad_out)
    return out


def kernel(molnodes, moledges, ringnodes, ringedges, f_t, params):
    p = params
    moln = molnodes.reshape(_B * _MOL_N, _NF)
    molE = moledges.transpose(0, 1, 3, 2) \
        .reshape(_B, _MOL_N, _EF * _MOL_N).astype(bf16)
    ringn = ringnodes.reshape(_B * _RING_N, _NF)
    ringE = ringedges.transpose(0, 1, 3, 2) \
        .reshape(_B, _RING_N, _EF * _RING_N).astype(bf16)

    seg_full = [('nodes', _RING_N * _F), ('rg', _G), ('mg', _G),
                ('ft', _NCLQ)]
    seg_term = [('rg', _G), ('ft', _NCLQ)]
    wts = (_prep_ggnn(p['molgnn']) + _prep_ggnn(p['ringgnn'])
           + _prep_mlp(p['add1']) + _prep_mlp(p['conn1'])
           + _prep_head(p['add2'], seg_full)
           + _prep_head(p['conn2'], seg_full)
           + _prep_head(p['term2'], seg_term, pad_out=8))

    bb = _BB
    grid = (_B // bb,)
    ops = [moln, molE, ringn, ringE, f_t] + wts
    in_specs = [
        pl.BlockSpec((bb * _MOL_N, _NF), lambda i: (i, 0)),
        pl.BlockSpec((bb, _MOL_N, _EF * _MOL_N), lambda i: (i, 0, 0)),
        pl.BlockSpec((bb * _RING_N, _NF), lambda i: (i, 0)),
        pl.BlockSpec((bb, _RING_N, _EF * _RING_N), lambda i: (i, 0, 0)),
        pl.BlockSpec((bb, _NCLQ), lambda i: (i, 0)),
    ] + [_const_spec(w) for w in wts]
    out_shape = [jax.ShapeDtypeStruct((_B, 128), f32),
                 jax.ShapeDtypeStruct((_B, 64), f32),
                 jax.ShapeDtypeStruct((_B, 8), f32)]
    out_specs = [pl.BlockSpec((bb, 128), lambda i: (i, 0)),
                 pl.BlockSpec((bb, 64), lambda i: (i, 0)),
                 pl.BlockSpec((bb, 8), lambda i: (i, 0))]
    add_out, conn_out, term_out = pl.pallas_call(
        functools.partial(_mega_body, bb),
        grid=grid,
        in_specs=in_specs,
        out_specs=out_specs,
        out_shape=out_shape,
    )(*ops)
    return jnp.concatenate([add_out, conn_out, term_out[:, :1]], axis=1)


# speedup vs baseline: 1.0775x; 1.0775x over previous
import functools

import jax
import jax.numpy as jnp
from jax.experimental import pallas as pl

_B = 1024
_MOL_N = 40
_RING_N = 12
_NF = 128
_EF = 4
_H = 128
_G = 128
_F = 16
_NCLQ = 64
_MLP_H = 512
_STEPS = 2

_BB_MOL = 32
_BB_RING = 32
_BB_HEADS = 256

_SELU_S = 1.0507009873554805
_SELU_A = 1.6732632423543772

bf16 = jnp.bfloat16
f32 = jnp.float32


def _selu(x):
    return _SELU_S * jnp.where(x > 0, x, _SELU_A * (jnp.exp(x) - 1.0))


def _dot(x, w):
    return jax.lax.dot_general(x, w, (((1,), (0,)), ((), ())),
                               preferred_element_type=f32)


def _mmk(x, wstk):
    xb = x.astype(bf16)
    return _dot(jnp.concatenate([xb, xb], axis=1), wstk)


def _stack_hilo(w):
    hi = w.astype(bf16)
    lo = (w - hi.astype(f32)).astype(bf16)
    return jnp.concatenate([hi, lo], axis=0)


def _mm2(xb, wcat, dout):
    y = _dot(xb, wcat)
    return y[:, :dout] + y[:, dout:]


def _hilo_lanes(x):
    hi = x.astype(bf16)
    lo = (x - hi.astype(f32)).astype(bf16)
    return jnp.concatenate([hi, lo], axis=-1)


def _cat_hilo(w):
    hi = w.astype(bf16)
    lo = (w - hi.astype(f32)).astype(bf16)
    return jnp.concatenate([hi, lo], axis=1)


def _const_spec(arr):
    nd = arr.ndim
    return pl.BlockSpec(arr.shape, lambda i, _nd=nd: (0,) * _nd)


def _ggnn_body(n, bb, pack, with_heads, nodes_ref, edges_ref,
               wemb_ref, bemb_ref, wmsg_ref, wzr_ref, bzr_ref,
               whc_ref, bhc_ref, gw_ref, gb_ref, ew_ref, eb_ref, *rest):
    if with_heads:
        (a0w, a0b, a1w, a1b, a2w, a2b,
         c0w, c0b, c1w, c1b, c2w, c2b,
         gout_ref, a1out_ref, c1out_ref) = rest
    else:
        (gout_ref,) = rest

    x = nodes_ref[...].astype(bf16)
    h = jnp.tanh(_mmk(x, wemb_ref[...]) + bemb_ref[...])

    ek = _EF * n
    e3 = edges_ref[...]
    if pack > 1:
        e4 = e3.reshape(bb // pack, pack, n, ek)
        bands = []
        for g in range(pack):
            pieces = []
            if g:
                pieces.append(jnp.zeros((bb // pack, n, g * ek), bf16))
            pieces.append(e4[:, g])
            if pack - 1 - g:
                pieces.append(
                    jnp.zeros((bb // pack, n, (pack - 1 - g) * ek), bf16))
            bands.append(jnp.concatenate(pieces, axis=2))
        eblk = jnp.concatenate(bands, axis=1)
    for _ in range(_STEPS):
        y = _mmk(h, wmsg_ref[...])
        cats = []
        for e in range(_EF):
            cats.append(_hilo_lanes(y[:, _H * e: _H * (e + 1)])
                        .reshape(bb, n, 2 * _H))
        whm3 = jnp.concatenate(cats, axis=1)
        ms = []
        if pack > 1:
            wblk = whm3.reshape(bb // pack, pack * ek, 2 * _H)
            for q in range(bb // pack):
                mq = _dot(eblk[q], wblk[q])
                ms.append(mq[:, :_H] + mq[:, _H:])
        else:
            for g in range(bb):
                mg = _dot(e3[g], whm3[g])
                ms.append(mg[:, :_H] + mg[:, _H:])
        m = jnp.concatenate(ms, axis=0)

        zr = jax.nn.sigmoid(
            _mmk(jnp.concatenate([m, h], axis=1), wzr_ref[...])
            + bzr_ref[...])
        z, r = zr[:, :_H], zr[:, _H:]
        x2 = jnp.concatenate([m, r * h], axis=1).astype(bf16)
        hc = jnp.tanh(_mm2(x2, whc_ref[...], _H) + bhc_ref[...])
        h = (1.0 - z) * h + z * hc

    hb = h.astype(bf16)
    gate = jax.nn.sigmoid(_dot(jnp.concatenate([hb, x], axis=1), gw_ref[...])
                          + gb_ref[...])
    emb = _dot(hb, ew_ref[...]) + eb_ref[...]
    pooled = (gate * emb).reshape(bb, n, _G).sum(axis=1)
    gout_ref[...] = pooled.astype(bf16)

    if with_heads:
        def head(w0, b0, w1, b1, w2, b2):
            t = _selu(_dot(hb, w0[...]) + b0[...])
            t = _selu(_dot(t.astype(bf16), w1[...]) + b1[...])
            return _dot(t.astype(bf16), w2[...]) + b2[...]
        a1out_ref[...] = head(a0w, a0b, a1w, a1b, a2w, a2b).astype(bf16) \
            .reshape(bb, n, _F)
        c1out_ref[...] = head(c0w, c0b, c1w, c1b, c2w, c2b).astype(bf16) \
            .reshape(bb, n, _F)


def _ggnn_call(n, bb, pack, nodes2d, edges3, wts, head_wts=None):
    grid = (_B // bb,)
    m = bb * n
    with_heads = head_wts is not None
    ops = [nodes2d, edges3] + list(wts) + (list(head_wts) if with_heads else [])
    in_specs = [
        pl.BlockSpec((m, _NF), lambda i: (i, 0)),
        pl.BlockSpec((bb, n, _EF * n), lambda i: (i, 0, 0)),
    ] + [_const_spec(w) for w in ops[2:]]
    out_shape = [jax.ShapeDtypeStruct((_B, _G), bf16)]
    out_specs = [pl.BlockSpec((bb, _G), lambda i: (i, 0))]
    if with_heads:
        out_shape += [jax.ShapeDtypeStruct((_B, n, _F), bf16)] * 2
        out_specs += [pl.BlockSpec((bb, n, _F), lambda i: (i, 0, 0))] * 2
    res = pl.pallas_call(
        functools.partial(_ggnn_body, n, bb, pack, with_heads),
        grid=grid,
        in_specs=in_specs,
        out_specs=out_specs,
        out_shape=out_shape,
    )(*ops)
    return res


def _heads_body(xa_ref, xc_ref, xt_ref,
                aw0, ab0, aw1, ab1, aw2, ab2,
                cw0, cb0, cw1, cb1, cw2, cb2,
                tw0, tb0, tw1, tb1, tw2, tb2,
                add_ref, conn_ref, term_ref):
    def mlp3(x, w0, b0, w1, b1, w2, b2):
        t = _selu(_dot(x, w0[...]) + b0[...])
        t = _selu(_dot(t.astype(bf16), w1[...]) + b1[...])
        return _dot(t.astype(bf16), w2[...]) + b2[...]
    add_ref[...] = mlp3(xa_ref[...], aw0, ab0, aw1, ab1, aw2, ab2)
    conn_ref[...] = mlp3(xc_ref[...], cw0, cb0, cw1, cb1, cw2, cb2)
    term_ref[...] = mlp3(xt_ref[...], tw0, tb0, tw1, tb1, tw2, tb2)


def _heads_call(xa, xc, xt, wts):
    bb = _BB_HEADS
    grid = (_B // bb,)
    ops = [xa, xc, xt] + list(wts)
    in_specs = [
        pl.BlockSpec((bb, xa.shape[1]), lambda i: (i, 0)),
        pl.BlockSpec((bb, xc.shape[1]), lambda i: (i, 0)),
        pl.BlockSpec((bb, xt.shape[1]), lambda i: (i, 0)),
    ] + [_const_spec(w) for w in wts]
    out_shape = [jax.ShapeDtypeStruct((_B, 128), f32),
                 jax.ShapeDtypeStruct((_B, 64), f32),
                 jax.ShapeDtypeStruct((_B, 8), f32)]
    out_specs = [pl.BlockSpec((bb, 128), lambda i: (i, 0)),
                 pl.BlockSpec((bb, 64), lambda i: (i, 0)),
                 pl.BlockSpec((bb, 8), lambda i: (i, 0))]
    return pl.pallas_call(
        _heads_body,
        grid=grid,
        in_specs=in_specs,
        out_specs=out_specs,
        out_shape=out_shape,
    )(*ops)


def _prep_ggnn(p):
    wemb = _stack_hilo(p['embed'][0])
    bemb = p['embed'][1].reshape(1, -1)
    wmsg = jnp.concatenate([_stack_hilo(p['msg_W'][e]) for e in range(_EF)],
                           axis=1)
    wzr_f = jnp.concatenate(
        [jnp.concatenate([p['Wz'][0], p['Uz'][0]], axis=0),
         jnp.concatenate([p['Wr'][0], p['Ur'][0]], axis=0)], axis=1)
    wzr = _stack_hilo(wzr_f)
    bzr = jnp.concatenate([p['Wz'][1] + p['Uz'][1],
                           p['Wr'][1] + p['Ur'][1]]).reshape(1, -1)
    whc = _cat_hilo(jnp.concatenate([p['Wh'][0], p['Uh'][0]], axis=0))
    bhc = (p['Wh'][1] + p['Uh'][1]).reshape(1, -1)
    gw = p['gate'][0].astype(bf16)
    gb = p['gate'][1].reshape(1, -1)
    ew = p['emb'][0].astype(bf16)
    eb = p['emb'][1].reshape(1, -1)
    return [wemb, bemb, wmsg, wzr, bzr, whc, bhc, gw, gb, ew, eb]


def _prep_mlp(layers, pad_out=None):
    out = []
    for i, (w, b) in enumerate(layers):
        if pad_out is not None and i == len(layers) - 1:
            w = jnp.pad(w, ((0, 0), (0, pad_out - w.shape[1])))
            b = jnp.pad(b, ((0, pad_out - b.shape[0]),))
        out += [w.astype(bf16), b.reshape(1, -1)]
    return out


def kernel(molnodes, moledges, ringnodes, ringedges, f_t, params):
    p = params
    moln = molnodes.reshape(_B * _MOL_N, _NF)
    molE = moledges.transpose(0, 1, 3, 2).reshape(_B, _MOL_N, _EF * _MOL_N) \
        .astype(bf16)
    ringn = ringnodes.reshape(_B * _RING_N, _NF)
    ringE = ringedges.transpose(0, 1, 3, 2) \
        .reshape(_B, _RING_N, _EF * _RING_N).astype(bf16)

    rg, a1n, c1n = _ggnn_call(
        _RING_N, _BB_RING, 4, ringn, ringE, _prep_ggnn(p['ringgnn']),
        _prep_mlp(p['add1']) + _prep_mlp(p['conn1']))
    (mg,) = _ggnn_call(_MOL_N, _BB_MOL, 1, moln, molE,
                       _prep_ggnn(p['molgnn']))

    ftb = f_t.astype(bf16)
    a1 = a1n.reshape(_B, _RING_N * _F)
    c1 = c1n.reshape(_B, _RING_N * _F)
    xa = jnp.concatenate([a1, rg, mg, ftb], axis=1)
    xc = jnp.concatenate([c1, rg, mg, ftb], axis=1)
    xt = jnp.concatenate([rg, ftb], axis=1)

    head_wts = (_prep_mlp(p['add2']) + _prep_mlp(p['conn2'])
                + _prep_mlp(p['term2'], pad_out=8))
    add_out, conn_out, term_out = _heads_call(xa, xc, xt, head_wts)
    return jnp.concatenate([add_out, conn_out, term_out[:, :1]], axis=1)


# speedup vs baseline: 1.1884x; 1.1029x over previous
import functools

import jax
import jax.numpy as jnp
from jax.experimental import pallas as pl

_B = 1024
_MOL_N = 40
_RING_N = 12
_NF = 128
_EF = 4
_H = 128
_G = 128
_F = 16
_NCLQ = 64
_MLP_H = 512
_STEPS = 2

_BB_MOL = 64
_BB_RING = 64
_BB_HEADS = 256

_SELU_S = 1.0507009873554805
_SELU_A = 1.6732632423543772

bf16 = jnp.bfloat16
f32 = jnp.float32


def _selu(x):
    return _SELU_S * jnp.where(x > 0, x, _SELU_A * (jnp.exp(x) - 1.0))


def _dot(x, w):
    return jax.lax.dot_general(x, w, (((1,), (0,)), ((), ())),
                               preferred_element_type=f32)


def _mmk(x, wstk):
    xb = x.astype(bf16)
    return _dot(jnp.concatenate([xb, xb], axis=1), wstk)


def _stack_hilo(w):
    hi = w.astype(bf16)
    lo = (w - hi.astype(f32)).astype(bf16)
    return jnp.concatenate([hi, lo], axis=0)


def _mm2(xb, wcat, dout):
    y = _dot(xb, wcat)
    return y[:, :dout] + y[:, dout:]


def _hilo_lanes(x):
    hi = x.astype(bf16)
    lo = (x - hi.astype(f32)).astype(bf16)
    return jnp.concatenate([hi, lo], axis=-1)


def _cat_hilo(w):
    hi = w.astype(bf16)
    lo = (w - hi.astype(f32)).astype(bf16)
    return jnp.concatenate([hi, lo], axis=1)


def _const_spec(arr):
    nd = arr.ndim
    return pl.BlockSpec(arr.shape, lambda i, _nd=nd: (0,) * _nd)


def _ggnn_body(n, bb, pack, with_heads, nodes_ref, edges_ref,
               wemb_ref, bemb_ref, wmsg_ref, wzr_ref, bzr_ref,
               whc_ref, bhc_ref, gw_ref, gb_ref, ew_ref, eb_ref, *rest):
    if with_heads:
        (a0w, a0b, a1w, a1b, a2w, a2b,
         c0w, c0b, c1w, c1b, c2w, c2b,
         gout_ref, a1out_ref, c1out_ref) = rest
    else:
        (gout_ref,) = rest

    x = nodes_ref[...].astype(bf16)
    h = jnp.tanh(_mmk(x, wemb_ref[...]) + bemb_ref[...])

    ek = _EF * n
    e3 = edges_ref[...]
    if pack > 1:
        e4 = e3.reshape(bb // pack, pack, n, ek)
        bands = []
        for g in range(pack):
            pieces = []
            if g:
                pieces.append(jnp.zeros((bb // pack, n, g * ek), bf16))
            pieces.append(e4[:, g])
            if pack - 1 - g:
                pieces.append(
                    jnp.zeros((bb // pack, n, (pack - 1 - g) * ek), bf16))
            bands.append(jnp.concatenate(pieces, axis=2))
        eblk = jnp.concatenate(bands, axis=1)
    for _ in range(_STEPS):
        y = _mmk(h, wmsg_ref[...])
        cats = []
        for e in range(_EF):
            cats.append(_hilo_lanes(y[:, _H * e: _H * (e + 1)])
                        .reshape(bb, n, 2 * _H))
        whm3 = jnp.concatenate(cats, axis=1)
        ms = []
        if pack > 1:
            wblk = whm3.reshape(bb // pack, pack * ek, 2 * _H)
            for q in range(bb // pack):
                mq = _dot(eblk[q], wblk[q])
                ms.append(mq[:, :_H] + mq[:, _H:])
        else:
            for g in range(bb):
                mg = _dot(e3[g], whm3[g])
                ms.append(mg[:, :_H] + mg[:, _H:])
        m = jnp.concatenate(ms, axis=0)

        zr = jax.nn.sigmoid(
            _mmk(jnp.concatenate([m, h], axis=1), wzr_ref[...])
            + bzr_ref[...])
        z, r = zr[:, :_H], zr[:, _H:]
        x2 = jnp.concatenate([m, r * h], axis=1).astype(bf16)
        hc = jnp.tanh(_mm2(x2, whc_ref[...], _H) + bhc_ref[...])
        h = (1.0 - z) * h + z * hc

    hb = h.astype(bf16)
    gate = jax.nn.sigmoid(_dot(jnp.concatenate([hb, x], axis=1), gw_ref[...])
                          + gb_ref[...])
    emb = _dot(hb, ew_ref[...]) + eb_ref[...]
    pooled = (gate * emb).reshape(bb, n, _G).sum(axis=1)
    gout_ref[...] = pooled.astype(bf16)

    if with_heads:
        def head(w0, b0, w1, b1, w2, b2):
            t = _selu(_dot(hb, w0[...]) + b0[...])
            t = _selu(_dot(t.astype(bf16), w1[...]) + b1[...])
            return _dot(t.astype(bf16), w2[...]) + b2[...]
        a1out_ref[...] = head(a0w, a0b, a1w, a1b, a2w, a2b).astype(bf16) \
            .reshape(bb, n, _F)
        c1out_ref[...] = head(c0w, c0b, c1w, c1b, c2w, c2b).astype(bf16) \
            .reshape(bb, n, _F)


def _ggnn_call(n, bb, pack, nodes2d, edges3, wts, head_wts=None):
    grid = (_B // bb,)
    m = bb * n
    with_heads = head_wts is not None
    ops = [nodes2d, edges3] + list(wts) + (list(head_wts) if with_heads else [])
    in_specs = [
        pl.BlockSpec((m, _NF), lambda i: (i, 0)),
        pl.BlockSpec((bb, n, _EF * n), lambda i: (i, 0, 0)),
    ] + [_const_spec(w) for w in ops[2:]]
    out_shape = [jax.ShapeDtypeStruct((_B, _G), bf16)]
    out_specs = [pl.BlockSpec((bb, _G), lambda i: (i, 0))]
    if with_heads:
        out_shape += [jax.ShapeDtypeStruct((_B, n, _F), bf16)] * 2
        out_specs += [pl.BlockSpec((bb, n, _F), lambda i: (i, 0, 0))] * 2
    res = pl.pallas_call(
        functools.partial(_ggnn_body, n, bb, pack, with_heads),
        grid=grid,
        in_specs=in_specs,
        out_specs=out_specs,
        out_shape=out_shape,
    )(*ops)
    return res


def _heads_body(xa_ref, xc_ref, xt_ref,
                aw0, ab0, aw1, ab1, aw2, ab2,
                cw0, cb0, cw1, cb1, cw2, cb2,
                tw0, tb0, tw1, tb1, tw2, tb2,
                add_ref, conn_ref, term_ref):
    def mlp3(x, w0, b0, w1, b1, w2, b2):
        t = _selu(_dot(x, w0[...]) + b0[...])
        t = _selu(_dot(t.astype(bf16), w1[...]) + b1[...])
        return _dot(t.astype(bf16), w2[...]) + b2[...]
    add_ref[...] = mlp3(xa_ref[...], aw0, ab0, aw1, ab1, aw2, ab2)
    conn_ref[...] = mlp3(xc_ref[...], cw0, cb0, cw1, cb1, cw2, cb2)
    term_ref[...] = mlp3(xt_ref[...], tw0, tb0, tw1, tb1, tw2, tb2)


def _heads_call(xa, xc, xt, wts):
    bb = _BB_HEADS
    grid = (_B // bb,)
    ops = [xa, xc, xt] + list(wts)
    in_specs = [
        pl.BlockSpec((bb, xa.shape[1]), lambda i: (i, 0)),
        pl.BlockSpec((bb, xc.shape[1]), lambda i: (i, 0)),
        pl.BlockSpec((bb, xt.shape[1]), lambda i: (i, 0)),
    ] + [_const_spec(w) for w in wts]
    out_shape = [jax.ShapeDtypeStruct((_B, 128), f32),
                 jax.ShapeDtypeStruct((_B, 64), f32),
                 jax.ShapeDtypeStruct((_B, 8), f32)]
    out_specs = [pl.BlockSpec((bb, 128), lambda i: (i, 0)),
                 pl.BlockSpec((bb, 64), lambda i: (i, 0)),
                 pl.BlockSpec((bb, 8), lambda i: (i, 0))]
    return pl.pallas_call(
        _heads_body,
        grid=grid,
        in_specs=in_specs,
        out_specs=out_specs,
        out_shape=out_shape,
    )(*ops)


def _prep_ggnn(p):
    wemb = _stack_hilo(p['embed'][0])
    bemb = p['embed'][1].reshape(1, -1)
    wmsg = jnp.concatenate([_stack_hilo(p['msg_W'][e]) for e in range(_EF)],
                           axis=1)
    wzr_f = jnp.concatenate(
        [jnp.concatenate([p['Wz'][0], p['Uz'][0]], axis=0),
         jnp.concatenate([p['Wr'][0], p['Ur'][0]], axis=0)], axis=1)
    wzr = _stack_hilo(wzr_f)
    bzr = jnp.concatenate([p['Wz'][1] + p['Uz'][1],
                           p['Wr'][1] + p['Ur'][1]]).reshape(1, -1)
    whc = _cat_hilo(jnp.concatenate([p['Wh'][0], p['Uh'][0]], axis=0))
    bhc = (p['Wh'][1] + p['Uh'][1]).reshape(1, -1)
    gw = p['gate'][0].astype(bf16)
    gb = p['gate'][1].reshape(1, -1)
    ew = p['emb'][0].astype(bf16)
    eb = p['emb'][1].reshape(1, -1)
    return [wemb, bemb, wmsg, wzr, bzr, whc, bhc, gw, gb, ew, eb]


def _prep_mlp(layers, pad_out=None):
    out = []
    for i, (w, b) in enumerate(layers):
        if pad_out is not None and i == len(layers) - 1:
            w = jnp.pad(w, ((0, 0), (0, pad_out - w.shape[1])))
            b = jnp.pad(b, ((0, pad_out - b.shape[0]),))
        out += [w.astype(bf16), b.reshape(1, -1)]
    return out


def kernel(molnodes, moledges, ringnodes, ringedges, f_t, params):
    p = params
    moln = molnodes.reshape(_B * _MOL_N, _NF)
    molE = moledges.transpose(0, 1, 3, 2).reshape(_B, _MOL_N, _EF * _MOL_N) \
        .astype(bf16)
    ringn = ringnodes.reshape(_B * _RING_N, _NF)
    ringE = ringedges.transpose(0, 1, 3, 2) \
        .reshape(_B, _RING_N, _EF * _RING_N).astype(bf16)

    rg, a1n, c1n = _ggnn_call(
        _RING_N, _BB_RING, 4, ringn, ringE, _prep_ggnn(p['ringgnn']),
        _prep_mlp(p['add1']) + _prep_mlp(p['conn1']))
    (mg,) = _ggnn_call(_MOL_N, _BB_MOL, 1, moln, molE,
                       _prep_ggnn(p['molgnn']))

    ftb = f_t.astype(bf16)
    a1 = a1n.reshape(_B, _RING_N * _F)
    c1 = c1n.reshape(_B, _RING_N * _F)
    xa = jnp.concatenate([a1, rg, mg, ftb], axis=1)
    xc = jnp.concatenate([c1, rg, mg, ftb], axis=1)
    xt = jnp.concatenate([rg, ftb], axis=1)

    head_wts = (_prep_mlp(p['add2']) + _prep_mlp(p['conn2'])
                + _prep_mlp(p['term2'], pad_out=8))
    add_out, conn_out, term_out = _heads_call(xa, xc, xt, head_wts)
    return jnp.concatenate([add_out, conn_out, term_out[:, :1]], axis=1)


# speedup vs baseline: 1.2222x; 1.0284x over previous
import functools

import jax
import jax.numpy as jnp
from jax.experimental import pallas as pl

_B = 1024
_MOL_N = 40
_RING_N = 12
_NF = 128
_EF = 4
_H = 128
_G = 128
_F = 16
_NCLQ = 64
_MLP_H = 512
_STEPS = 2

_BB_MOL = 128
_BB_RING = 128
_BB_HEADS = 256

_SELU_S = 1.0507009873554805
_SELU_A = 1.6732632423543772

bf16 = jnp.bfloat16
f32 = jnp.float32


def _selu(x):
    return _SELU_S * jnp.where(x > 0, x, _SELU_A * (jnp.exp(x) - 1.0))


def _dot(x, w):
    return jax.lax.dot_general(x, w, (((1,), (0,)), ((), ())),
                               preferred_element_type=f32)


def _mmk(x, wstk):
    xb = x.astype(bf16)
    return _dot(jnp.concatenate([xb, xb], axis=1), wstk)


def _stack_hilo(w):
    hi = w.astype(bf16)
    lo = (w - hi.astype(f32)).astype(bf16)
    return jnp.concatenate([hi, lo], axis=0)


def _mm2(xb, wcat, dout):
    y = _dot(xb, wcat)
    return y[:, :dout] + y[:, dout:]


def _hilo_lanes(x):
    hi = x.astype(bf16)
    lo = (x - hi.astype(f32)).astype(bf16)
    return jnp.concatenate([hi, lo], axis=-1)


def _cat_hilo(w):
    hi = w.astype(bf16)
    lo = (w - hi.astype(f32)).astype(bf16)
    return jnp.concatenate([hi, lo], axis=1)


def _const_spec(arr):
    nd = arr.ndim
    return pl.BlockSpec(arr.shape, lambda i, _nd=nd: (0,) * _nd)


def _ggnn_body(n, bb, pack, with_heads, nodes_ref, edges_ref,
               wemb_ref, bemb_ref, wmsg_ref, wzr_ref, bzr_ref,
               whc_ref, bhc_ref, gw_ref, gb_ref, ew_ref, eb_ref, *rest):
    if with_heads:
        (a0w, a0b, a1w, a1b, a2w, a2b,
         c0w, c0b, c1w, c1b, c2w, c2b,
         gout_ref, a1out_ref, c1out_ref) = rest
    else:
        (gout_ref,) = rest

    x = nodes_ref[...].astype(bf16)
    h = jnp.tanh(_mmk(x, wemb_ref[...]) + bemb_ref[...])

    ek = _EF * n
    e3 = edges_ref[...]
    if pack > 1:
        e4 = e3.reshape(bb // pack, pack, n, ek)
        bands = []
        for g in range(pack):
            pieces = []
            if g:
                pieces.append(jnp.zeros((bb // pack, n, g * ek), bf16))
            pieces.append(e4[:, g])
            if pack - 1 - g:
                pieces.append(
                    jnp.zeros((bb // pack, n, (pack - 1 - g) * ek), bf16))
            bands.append(jnp.concatenate(pieces, axis=2))
        eblk = jnp.concatenate(bands, axis=1)
    for _ in range(_STEPS):
        y = _mmk(h, wmsg_ref[...])
        cats = []
        for e in range(_EF):
            cats.append(_hilo_lanes(y[:, _H * e: _H * (e + 1)])
                        .reshape(bb, n, 2 * _H))
        whm3 = jnp.concatenate(cats, axis=1)
        ms = []
        if pack > 1:
            wblk = whm3.reshape(bb // pack, pack * ek, 2 * _H)
            for q in range(bb // pack):
                mq = _dot(eblk[q], wblk[q])
                ms.append(mq[:, :_H] + mq[:, _H:])
        else:
            for g in range(bb):
                mg = _dot(e3[g], whm3[g])
                ms.append(mg[:, :_H] + mg[:, _H:])
        m = jnp.concatenate(ms, axis=0)

        zr = jax.nn.sigmoid(
            _mmk(jnp.concatenate([m, h], axis=1), wzr_ref[...])
            + bzr_ref[...])
        z, r = zr[:, :_H], zr[:, _H:]
        x2 = jnp.concatenate([m, r * h], axis=1).astype(bf16)
        hc = jnp.tanh(_mm2(x2, whc_ref[...], _H) + bhc_ref[...])
        h = (1.0 - z) * h + z * hc

    hb = h.astype(bf16)
    gate = jax.nn.sigmoid(_dot(jnp.concatenate([hb, x], axis=1), gw_ref[...])
                          + gb_ref[...])
    emb = _dot(hb, ew_ref[...]) + eb_ref[...]
    pooled = (gate * emb).reshape(bb, n, _G).sum(axis=1)
    gout_ref[...] = pooled.astype(bf16)

    if with_heads:
        def head(w0, b0, w1, b1, w2, b2):
            t = _selu(_dot(hb, w0[...]) + b0[...])
            t = _selu(_dot(t.astype(bf16), w1[...]) + b1[...])
            return _dot(t.astype(bf16), w2[...]) + b2[...]
        a1out_ref[...] = head(a0w, a0b, a1w, a1b, a2w, a2b).astype(bf16) \
            .reshape(bb, n, _F)
        c1out_ref[...] = head(c0w, c0b, c1w, c1b, c2w, c2b).astype(bf16) \
            .reshape(bb, n, _F)


def _ggnn_call(n, bb, pack, nodes2d, edges3, wts, head_wts=None):
    grid = (_B // bb,)
    m = bb * n
    with_heads = head_wts is not None
    ops = [nodes2d, edges3] + list(wts) + (list(head_wts) if with_heads else [])
    in_specs = [
        pl.BlockSpec((m, _NF), lambda i: (i, 0)),
        pl.BlockSpec((bb, n, _EF * n), lambda i: (i, 0, 0)),
    ] + [_const_spec(w) for w in ops[2:]]
    out_shape = [jax.ShapeDtypeStruct((_B, _G), bf16)]
    out_specs = [pl.BlockSpec((bb, _G), lambda i: (i, 0))]
    if with_heads:
        out_shape += [jax.ShapeDtypeStruct((_B, n, _F), bf16)] * 2
        out_specs += [pl.BlockSpec((bb, n, _F), lambda i: (i, 0, 0))] * 2
    res = pl.pallas_call(
        functools.partial(_ggnn_body, n, bb, pack, with_heads),
        grid=grid,
        in_specs=in_specs,
        out_specs=out_specs,
        out_shape=out_shape,
    )(*ops)
    return res


def _heads_body(xa_ref, xc_ref, xt_ref,
                aw0, ab0, aw1, ab1, aw2, ab2,
                cw0, cb0, cw1, cb1, cw2, cb2,
                tw0, tb0, tw1, tb1, tw2, tb2,
                add_ref, conn_ref, term_ref):
    def mlp3(x, w0, b0, w1, b1, w2, b2):
        t = _selu(_dot(x, w0[...]) + b0[...])
        t = _selu(_dot(t.astype(bf16), w1[...]) + b1[...])
        return _dot(t.astype(bf16), w2[...]) + b2[...]
    add_ref[...] = mlp3(xa_ref[...], aw0, ab0, aw1, ab1, aw2, ab2)
    conn_ref[...] = mlp3(xc_ref[...], cw0, cb0, cw1, cb1, cw2, cb2)
    term_ref[...] = mlp3(xt_ref[...], tw0, tb0, tw1, tb1, tw2, tb2)


def _heads_call(xa, xc, xt, wts):
    bb = _BB_HEADS
    grid = (_B // bb,)
    ops = [xa, xc, xt] + list(wts)
    in_specs = [
        pl.BlockSpec((bb, xa.shape[1]), lambda i: (i, 0)),
        pl.BlockSpec((bb, xc.shape[1]), lambda i: (i, 0)),
        pl.BlockSpec((bb, xt.shape[1]), lambda i: (i, 0)),
    ] + [_const_spec(w) for w in wts]
    out_shape = [jax.ShapeDtypeStruct((_B, 128), f32),
                 jax.ShapeDtypeStruct((_B, 64), f32),
                 jax.ShapeDtypeStruct((_B, 8), f32)]
    out_specs = [pl.BlockSpec((bb, 128), lambda i: (i, 0)),
                 pl.BlockSpec((bb, 64), lambda i: (i, 0)),
                 pl.BlockSpec((bb, 8), lambda i: (i, 0))]
    return pl.pallas_call(
        _heads_body,
        grid=grid,
        in_specs=in_specs,
        out_specs=out_specs,
        out_shape=out_shape,
    )(*ops)


def _prep_ggnn(p):
    wemb = _stack_hilo(p['embed'][0])
    bemb = p['embed'][1].reshape(1, -1)
    wmsg = jnp.concatenate([_stack_hilo(p['msg_W'][e]) for e in range(_EF)],
                           axis=1)
    wzr_f = jnp.concatenate(
        [jnp.concatenate([p['Wz'][0], p['Uz'][0]], axis=0),
         jnp.concatenate([p['Wr'][0], p['Ur'][0]], axis=0)], axis=1)
    wzr = _stack_hilo(wzr_f)
    bzr = jnp.concatenate([p['Wz'][1] + p['Uz'][1],
                           p['Wr'][1] + p['Ur'][1]]).reshape(1, -1)
    whc = _cat_hilo(jnp.concatenate([p['Wh'][0], p['Uh'][0]], axis=0))
    bhc = (p['Wh'][1] + p['Uh'][1]).reshape(1, -1)
    gw = p['gate'][0].astype(bf16)
    gb = p['gate'][1].reshape(1, -1)
    ew = p['emb'][0].astype(bf16)
    eb = p['emb'][1].reshape(1, -1)
    return [wemb, bemb, wmsg, wzr, bzr, whc, bhc, gw, gb, ew, eb]


def _prep_mlp(layers, pad_out=None):
    out = []
    for i, (w, b) in enumerate(layers):
        if pad_out is not None and i == len(layers) - 1:
            w = jnp.pad(w, ((0, 0), (0, pad_out - w.shape[1])))
            b = jnp.pad(b, ((0, pad_out - b.shape[0]),))
        out += [w.astype(bf16), b.reshape(1, -1)]
    return out


def kernel(molnodes, moledges, ringnodes, ringedges, f_t, params):
    p = params
    moln = molnodes.reshape(_B * _MOL_N, _NF)
    molE = moledges.transpose(0, 1, 3, 2).reshape(_B, _MOL_N, _EF * _MOL_N) \
        .astype(bf16)
    ringn = ringnodes.reshape(_B * _RING_N, _NF)
    ringE = ringedges.transpose(0, 1, 3, 2) \
        .reshape(_B, _RING_N, _EF * _RING_N).astype(bf16)

    rg, a1n, c1n = _ggnn_call(
        _RING_N, _BB_RING, 4, ringn, ringE, _prep_ggnn(p['ringgnn']),
        _prep_mlp(p['add1']) + _prep_mlp(p['conn1']))
    (mg,) = _ggnn_call(_MOL_N, _BB_MOL, 1, moln, molE,
                       _prep_ggnn(p['molgnn']))

    ftb = f_t.astype(bf16)
    a1 = a1n.reshape(_B, _RING_N * _F)
    c1 = c1n.reshape(_B, _RING_N * _F)
    xa = jnp.concatenate([a1, rg, mg, ftb], axis=1)
    xc = jnp.concatenate([c1, rg, mg, ftb], axis=1)
    xt = jnp.concatenate([rg, ftb], axis=1)

    head_wts = (_prep_mlp(p['add2']) + _prep_mlp(p['conn2'])
                + _prep_mlp(p['term2'], pad_out=8))
    add_out, conn_out, term_out = _heads_call(xa, xc, xt, head_wts)
    return jnp.concatenate([add_out, conn_out, term_out[:, :1]], axis=1)
